# Initial kernel scaffold; baseline (speedup 1.0000x reference)
import jax
import jax.numpy as jnp
from jax.experimental import pallas as pl


def kernel(x, seg, token_table, pos_table, seg_table, gamma, beta):
    raise NotImplementedError("write your pallas kernel here")


# speedup vs baseline: 2.9435x; 2.9435x over previous
import jax
import jax.numpy as jnp
from jax import lax
from jax.experimental import pallas as pl
from jax.experimental.pallas import tpu as pltpu
from jax.experimental.pallas import tpu_sc as plsc

B, S, V, D, NSEG = 1024, 200, 100000, 128, 2
EPS = 1e-5
NC, NS, L = 2, 16, 16
NW = NC * NS
CHUNK = 128
NCHUNK = B * S // CHUNK
CH_PER_W = NCHUNK // NW
NJ = D // L


def _rsqrt(x):
    i = lax.bitcast_convert_type(x, jnp.int32)
    y = lax.bitcast_convert_type(0x5F3759DF - (i >> 1), jnp.float32)
    for _ in range(3):
        y = y * (1.5 - 0.5 * x * y * y)
    return y


def _allsum(v):
    for m in (8, 4, 2, 1):
        v = v + jnp.take_along_axis(v, lax.iota(jnp.int32, 16) ^ m, axis=0)
    return v


def _sc_body(x_hbm, seg_hbm, tok_hbm, posseg_hbm, gb_hbm, out_hbm,
             posseg_v, rows_v, idx_v, segb_v, gb_v, sem):
    wid = lax.axis_index("s") * NC + lax.axis_index("c")

    pltpu.sync_copy(posseg_hbm, posseg_v)
    pltpu.sync_copy(gb_hbm, gb_v)
    gam = [gb_v[0, pl.ds(16 * j, 16)] for j in range(NJ)]
    bet = [gb_v[1, pl.ds(16 * j, 16)] for j in range(NJ)]

    @pl.loop(0, CH_PER_W)
    def _chunk(k):
        c = wid * CH_PER_W + k
        base = c * CHUNK
        pltpu.sync_copy(x_hbm.at[c], idx_v)
        pltpu.sync_copy(seg_hbm.at[c], segb_v)
        pltpu.async_copy(tok_hbm.at[idx_v], rows_v, sem).wait()

        @pl.loop(0, CHUNK // L)
        def _grp(g):
            sv = segb_v[pl.ds(g * L, L)]
            for u in range(L):
                i = g * L + u
                s_pos = lax.rem(base + i, S)
                pr = sv[u] * S + s_pos
                h = [rows_v[i, pl.ds(16 * j, 16)] + posseg_v[pr, pl.ds(16 * j, 16)]
                     for j in range(NJ)]
                s1 = ((h[0] + h[1]) + (h[2] + h[3])) + ((h[4] + h[5]) + (h[6] + h[7]))
                q = [v * v for v in h]
                s2 = ((q[0] + q[1]) + (q[2] + q[3])) + ((q[4] + q[5]) + (q[6] + q[7]))
                mean_v = _allsum(s1) * (1.0 / D)
                var_v = _allsum(s2) * (1.0 / D) - mean_v * mean_v
                inv_v = _rsqrt(var_v + EPS)
                mi_v = mean_v * inv_v
                for j in range(NJ):
                    rows_v[i, pl.ds(16 * j, 16)] = (h[j] * inv_v - mi_v) * gam[j] + bet[j]

        pltpu.sync_copy(rows_v, out_hbm.at[pl.ds(base, CHUNK)])


@jax.jit
def _run(x2, seg2, token_table, posseg, gb):
    mesh = plsc.VectorSubcoreMesh(core_axis_name="c", subcore_axis_name="s")
    return pl.kernel(
        _sc_body,
        out_type=jax.ShapeDtypeStruct((B * S, D), jnp.float32),
        mesh=mesh,
        scratch_types=[
            pltpu.VMEM((NSEG * S, D), jnp.float32),
            pltpu.VMEM((CHUNK, D), jnp.float32),
            pltpu.VMEM((CHUNK,), jnp.int32),
            pltpu.VMEM((CHUNK,), jnp.int32),
            pltpu.VMEM((2, D), jnp.float32),
            pltpu.SemaphoreType.DMA,
        ],
    )(x2, seg2, token_table, posseg, gb)


def kernel(x, seg, token_table, pos_table, seg_table, gamma, beta):
    x2 = x.astype(jnp.int32).reshape(NCHUNK, CHUNK)
    seg2 = seg.astype(jnp.int32).reshape(NCHUNK, CHUNK)
    posseg = (seg_table[:, None, :] + pos_table[None, :, :]).reshape(NSEG * S, D)
    gb = jnp.stack([gamma, beta])
    out = _run(x2, seg2, token_table, posseg, gb)
    return out.reshape(B, S, D)


# speedup vs baseline: 3.4825x; 1.1831x over previous
import jax
import jax.numpy as jnp
from jax import lax
from jax.experimental import pallas as pl
from jax.experimental.pallas import tpu as pltpu
from jax.experimental.pallas import tpu_sc as plsc

B, S, V, D, NSEG = 1024, 200, 100000, 128, 2
EPS = 1e-5
NC, NS, L = 2, 16, 16
NW = NC * NS
CHUNK = 128
NCHUNK = B * S // CHUNK
CH_PER_W = NCHUNK // NW
NJ = D // L


def _rsqrt(x):
    i = lax.bitcast_convert_type(x, jnp.int32)
    y = lax.bitcast_convert_type(0x5F3759DF - (i >> 1), jnp.float32)
    for _ in range(3):
        y = y * (1.5 - 0.5 * x * y * y)
    return y


def _allsum(v):
    for m in (8, 4, 2, 1):
        v = v + jnp.take_along_axis(v, lax.iota(jnp.int32, 16) ^ m, axis=0)
    return v


def _sc_body(x_hbm, seg_hbm, tok_hbm, posseg_hbm, gb_hbm, out_hbm,
             posseg_v, rows0, rows1, outb0, outb1, idx0, idx1, seg0, seg1,
             gb_v, sg0, sg1, so0, so1):
    wid = lax.axis_index("s") * NC + lax.axis_index("c")

    pltpu.sync_copy(posseg_hbm, posseg_v)
    pltpu.sync_copy(gb_hbm, gb_v)
    gam = [gb_v[0, pl.ds(16 * j, 16)] for j in range(NJ)]
    bet = [gb_v[1, pl.ds(16 * j, 16)] for j in range(NJ)]

    def compute(rows_v, segb_v, base, out_v):
        @pl.loop(0, CHUNK // L)
        def _grp(g):
            sv = segb_v[pl.ds(g * L, L)]
            for u in range(L):
                i = g * L + u
                s_pos = lax.rem(base + i, S)
                pr = sv[u] * S + s_pos
                h = [rows_v[i, pl.ds(16 * j, 16)] + posseg_v[pr, pl.ds(16 * j, 16)]
                     for j in range(NJ)]
                s1 = ((h[0] + h[1]) + (h[2] + h[3])) + ((h[4] + h[5]) + (h[6] + h[7]))
                q = [v * v for v in h]
                s2 = ((q[0] + q[1]) + (q[2] + q[3])) + ((q[4] + q[5]) + (q[6] + q[7]))
                mean_v = _allsum(s1) * (1.0 / D)
                var_v = _allsum(s2) * (1.0 / D) - mean_v * mean_v
                inv_v = _rsqrt(var_v + EPS)
                mi_v = mean_v * inv_v
                for j in range(NJ):
                    out_v[i, pl.ds(16 * j, 16)] = (h[j] * inv_v - mi_v) * gam[j] + bet[j]

    c0 = wid * CH_PER_W
    pltpu.sync_copy(x_hbm.at[c0], idx0)
    pltpu.sync_copy(seg_hbm.at[c0], seg0)
    pltpu.async_copy(tok_hbm.at[idx0], rows0, sg0)

    @pl.loop(0, CH_PER_W // 2)
    def _pair(t):
        c = wid * CH_PER_W + 2 * t
        pltpu.make_async_copy(tok_hbm.at[idx0], rows0, sg0).wait()
        pltpu.sync_copy(x_hbm.at[c + 1], idx1)
        pltpu.sync_copy(seg_hbm.at[c + 1], seg1)
        pltpu.async_copy(tok_hbm.at[idx1], rows1, sg1)

        @pl.when(t > 0)
        def _():
            pltpu.make_async_copy(outb0, out_hbm.at[pl.ds(0, CHUNK)], so0).wait()

        compute(rows0, seg0, c * CHUNK, outb0)
        pltpu.async_copy(outb0, out_hbm.at[pl.ds(c * CHUNK, CHUNK)], so0)

        pltpu.make_async_copy(tok_hbm.at[idx1], rows1, sg1).wait()

        @pl.when(t + 1 < CH_PER_W // 2)
        def _():
            pltpu.sync_copy(x_hbm.at[c + 2], idx0)
            pltpu.sync_copy(seg_hbm.at[c + 2], seg0)
            pltpu.async_copy(tok_hbm.at[idx0], rows0, sg0)

        @pl.when(t > 0)
        def _():
            pltpu.make_async_copy(outb1, out_hbm.at[pl.ds(0, CHUNK)], so1).wait()

        compute(rows1, seg1, (c + 1) * CHUNK, outb1)
        pltpu.async_copy(outb1, out_hbm.at[pl.ds((c + 1) * CHUNK, CHUNK)], so1)

    pltpu.make_async_copy(outb0, out_hbm.at[pl.ds(0, CHUNK)], so0).wait()
    pltpu.make_async_copy(outb1, out_hbm.at[pl.ds(0, CHUNK)], so1).wait()


@jax.jit
def _run(x2, seg2, token_table, posseg, gb):
    mesh = plsc.VectorSubcoreMesh(core_axis_name="c", subcore_axis_name="s")
    return pl.kernel(
        _sc_body,
        out_type=jax.ShapeDtypeStruct((B * S, D), jnp.float32),
        mesh=mesh,
        scratch_types=[
            pltpu.VMEM((NSEG * S, D), jnp.float32),
            pltpu.VMEM((CHUNK, D), jnp.float32),
            pltpu.VMEM((CHUNK, D), jnp.float32),
            pltpu.VMEM((CHUNK, D), jnp.float32),
            pltpu.VMEM((CHUNK, D), jnp.float32),
            pltpu.VMEM((CHUNK,), jnp.int32),
            pltpu.VMEM((CHUNK,), jnp.int32),
            pltpu.VMEM((CHUNK,), jnp.int32),
            pltpu.VMEM((CHUNK,), jnp.int32),
            pltpu.VMEM((2, D), jnp.float32),
            pltpu.SemaphoreType.DMA,
            pltpu.SemaphoreType.DMA,
            pltpu.SemaphoreType.DMA,
            pltpu.SemaphoreType.DMA,
        ],
    )(x2, seg2, token_table, posseg, gb)


def kernel(x, seg, token_table, pos_table, seg_table, gamma, beta):
    x2 = x.astype(jnp.int32).reshape(NCHUNK, CHUNK)
    seg2 = seg.astype(jnp.int32).reshape(NCHUNK, CHUNK)
    posseg = (seg_table[:, None, :] + pos_table[None, :, :]).reshape(NSEG * S, D)
    gb = jnp.stack([gamma, beta])
    out = _run(x2, seg2, token_table, posseg, gb)
    return out.reshape(B, S, D)
